# baseline (speedup 1.0000x reference)
import functools

import jax
import jax.numpy as jnp
from jax import lax
from jax.experimental import pallas as pl
from jax.experimental.pallas import tpu as pltpu
from jax.experimental.pallas import tpu_sc as plsc

def _acosh(z):
    return jnp.log(z + jnp.sqrt((z - 1.0) * (z + 1.0)))


N_USERS = 2000
N_ITEMS = 8000
N_NODES = 10000
D = 128
N_EDGES = 320000
N_PAIRS = 4096

PAD_NODES = 10112
NW = 32
EG = 128
GPT = 80
CH = 40
NCH = GPT // CH
TOT_GROUPS = NW * GPT
NE_PAD = TOT_GROUPS * EG
ROWS_PER_TILE = PAD_NODES // 16


def _encode_body(x_ref, w_ref, m_ref):
    x = x_ref[...]
    lane = lax.broadcasted_iota(jnp.int32, x.shape, 1)
    is0 = lane == 0
    xs = jnp.where(is0, 0.0, x)
    s = jnp.sum(xs * xs, axis=-1, keepdims=True)
    x0 = jnp.sqrt(1.0 + s)
    dd = _acosh(jnp.maximum(x0, 1.0 + 1e-7))
    nrm = jnp.maximum(jnp.sqrt(s), 1e-7)
    t = (dd / nrm) * xs
    m = jnp.dot(t, w_ref[...], preferred_element_type=jnp.float32)
    m_ref[...] = jnp.where(is0, 1.0, m)


def _encode(x, w):
    return pl.pallas_call(
        _encode_body,
        out_shape=jax.ShapeDtypeStruct((PAD_NODES, D), jnp.float32),
    )(x, w)


def _edge_agg(m, idx, zeros):
    mesh = plsc.VectorSubcoreMesh(core_axis_name="c", subcore_axis_name="s")

    @functools.partial(
        pl.kernel,
        mesh=mesh,
        out_type=jax.ShapeDtypeStruct((2, PAD_NODES, D), jnp.float32),
        scratch_types=[
            pltpu.VMEM((2, CH, EG), jnp.int32),
            pltpu.VMEM((EG, D), jnp.float32),
            pltpu.VMEM((EG, D), jnp.float32),
            pltpu.VMEM_SHARED((PAD_NODES, D), jnp.float32),
            pltpu.SemaphoreType.DMA,
            pltpu.SemaphoreType.DMA,
        ],
    )
    def body(m_hbm, idx_hbm, zero_hbm, out_hbm,
             idx_v, rows0_v, rows1_v, agg_sh, sem0, sem1):
        c = lax.axis_index("c")
        s = lax.axis_index("s")
        wid = s * 2 + c
        rows = (rows0_v, rows1_v)
        sems = (sem0, sem1)

        pltpu.sync_copy(zero_hbm.at[pl.ds(s * ROWS_PER_TILE, ROWS_PER_TILE)],
                        agg_sh.at[pl.ds(s * ROWS_PER_TILE, ROWS_PER_TILE)])
        plsc.subcore_barrier()

        def chunk_body(ch, carry):
            pltpu.sync_copy(idx_hbm.at[wid * NCH + ch], idx_v)
            pltpu.async_copy(m_hbm.at[idx_v.at[0, 0]], rows0_v, sem0)
            pltpu.async_copy(m_hbm.at[idx_v.at[0, 1]], rows1_v, sem1)

            def step(i, carry2):
                g = 2 * i
                for b in range(2):
                    pltpu.make_async_copy(
                        m_hbm.at[idx_v.at[0, g + b]], rows[b], sems[b]).wait()
                    pltpu.sync_copy(rows[b], agg_sh.at[idx_v.at[1, g + b]],
                                    add=True)

                    @pl.when(g + b + 2 < CH)
                    def _():
                        pltpu.async_copy(
                            m_hbm.at[idx_v.at[0, g + b + 2]], rows[b], sems[b])
                return carry2

            lax.fori_loop(0, CH // 2, step, 0)
            return carry

        lax.fori_loop(0, NCH, chunk_body, 0)

        plsc.subcore_barrier()
        pltpu.sync_copy(agg_sh.at[pl.ds(s * ROWS_PER_TILE, ROWS_PER_TILE)],
                        out_hbm.at[c].at[pl.ds(s * ROWS_PER_TILE, ROWS_PER_TILE)])

    return body(m, idx, zeros)


def _pair_gather(m, agg2, pu, pv):
    mesh = plsc.VectorSubcoreMesh(core_axis_name="c", subcore_axis_name="s")

    @functools.partial(
        pl.kernel,
        mesh=mesh,
        out_type=jax.ShapeDtypeStruct((2, 3, N_PAIRS, D), jnp.float32),
        scratch_types=[
            pltpu.VMEM((EG,), jnp.int32),
            pltpu.VMEM((EG,), jnp.int32),
            pltpu.VMEM((EG, D), jnp.float32),
            pltpu.VMEM((EG, D), jnp.float32),
            pltpu.VMEM((EG, D), jnp.float32),
            pltpu.VMEM((EG, D), jnp.float32),
            pltpu.VMEM((EG, D), jnp.float32),
            pltpu.VMEM((EG, D), jnp.float32),
            pltpu.SemaphoreType.DMA,
        ],
    )
    def body(m_hbm, agg_hbm, pu_hbm, pv_hbm, out_hbm,
             idxu_v, idxv_v, b0, b1, b2, b3, b4, b5, sem):
        c = lax.axis_index("c")
        s = lax.axis_index("s")
        wid = s * 2 + c
        off = wid * EG
        pltpu.sync_copy(pu_hbm.at[pl.ds(off, EG)], idxu_v)
        pltpu.sync_copy(pv_hbm.at[pl.ds(off, EG)], idxv_v)
        plan = (
            (m_hbm, idxu_v, b0, 0, 0), (agg_hbm.at[0], idxu_v, b1, 0, 1),
            (agg_hbm.at[1], idxu_v, b2, 0, 2), (m_hbm, idxv_v, b3, 1, 0),
            (agg_hbm.at[0], idxv_v, b4, 1, 1), (agg_hbm.at[1], idxv_v, b5, 1, 2),
        )
        for tbl, idx, buf, _, _k in plan:
            pltpu.async_copy(tbl.at[idx], buf, sem)
        for tbl, idx, buf, _, _k in plan:
            pltpu.make_async_copy(tbl.at[idx], buf, sem).wait()
        for _, _, buf, side, k in plan:
            pltpu.sync_copy(buf, out_hbm.at[side, k].at[pl.ds(off, EG)])

    return body(m, agg2, pu, pv)


def _dist_body(g_ref, o_ref):
    lane = lax.broadcasted_iota(jnp.int32, (N_PAIRS, D), 1)
    is0 = lane == 0

    def node_h(side):
        m = g_ref[side, 0]
        a = g_ref[side, 1] + g_ref[side, 2]
        deg = jnp.sum(jnp.where(is0, a, 0.0), axis=-1, keepdims=True)
        deg = jnp.maximum(deg, 1.0)
        vs = jnp.where(is0, 0.0, m + a / deg)
        n = jnp.maximum(
            jnp.sqrt(jnp.sum(vs * vs, axis=-1, keepdims=True)), 1e-7)
        sinh_n = 0.5 * (jnp.exp(n) - jnp.exp(-n))
        xs = (sinh_n / n) * vs
        x0 = jnp.sqrt(1.0 + jnp.sum(xs * xs, axis=-1, keepdims=True))
        return jnp.where(is0, x0, xs)

    p = node_h(0) * node_h(1)
    p0 = jnp.sum(jnp.where(is0, p, 0.0), axis=-1, keepdims=True)
    rest = jnp.sum(jnp.where(is0, 0.0, p), axis=-1, keepdims=True)
    neg_inner = p0 - rest
    d = _acosh(jnp.maximum(neg_inner, 1.0 + 1e-5))
    o_ref[...] = jnp.broadcast_to(jnp.minimum(d * d, 15.0), p.shape)


def _dist(gath):
    return pl.pallas_call(
        _dist_body,
        out_shape=jax.ShapeDtypeStruct((N_PAIRS, D), jnp.float32),
    )(gath)


def kernel(utg, vtg, W, edge_index, pair_idx):
    x = jnp.concatenate(
        [utg, vtg, jnp.zeros((PAD_NODES - N_NODES, D), jnp.float32)], axis=0)
    m = _encode(x, W)

    n_pad_e = NE_PAD - N_EDGES
    pad = (N_NODES
           + jnp.arange(n_pad_e, dtype=jnp.int32) % (PAD_NODES - N_NODES))
    src = jnp.concatenate([edge_index[0].astype(jnp.int32), pad])
    dst = jnp.concatenate([edge_index[1].astype(jnp.int32), pad])
    idx = jnp.stack([src.reshape(NW * NCH, CH, EG),
                     dst.reshape(NW * NCH, CH, EG)], axis=1)
    zeros = jnp.zeros((PAD_NODES, D), jnp.float32)
    agg2 = _edge_agg(m, idx, zeros)

    pu = pair_idx[:, 0].astype(jnp.int32)
    pv = pair_idx[:, 1].astype(jnp.int32)
    gath = _pair_gather(m, agg2, pu, pv)

    out = _dist(gath)
    return out[:, :1]


# speedup vs baseline: 1.0070x; 1.0070x over previous
import functools

import jax
import jax.numpy as jnp
from jax import lax
from jax.experimental import pallas as pl
from jax.experimental.pallas import tpu as pltpu
from jax.experimental.pallas import tpu_sc as plsc

def _acosh(z):
    return jnp.log(z + jnp.sqrt((z - 1.0) * (z + 1.0)))


N_USERS = 2000
N_ITEMS = 8000
N_NODES = 10000
D = 128
N_EDGES = 320000
N_PAIRS = 4096

PAD_NODES = 10112
NW = 32
EG = 128
GPT = 80
CH = 40
NCH = GPT // CH
TOT_GROUPS = NW * GPT
NE_PAD = TOT_GROUPS * EG
ROWS_PER_TILE = PAD_NODES // 16


def _encode_body(x_ref, w_ref, m_ref):
    x = x_ref[...]
    lane = lax.broadcasted_iota(jnp.int32, x.shape, 1)
    is0 = lane == 0
    xs = jnp.where(is0, 0.0, x)
    s = jnp.sum(xs * xs, axis=-1, keepdims=True)
    x0 = jnp.sqrt(1.0 + s)
    dd = _acosh(jnp.maximum(x0, 1.0 + 1e-7))
    nrm = jnp.maximum(jnp.sqrt(s), 1e-7)
    t = (dd / nrm) * xs
    m = jnp.dot(t, w_ref[...], preferred_element_type=jnp.float32)
    m_ref[...] = jnp.where(is0, 1.0, m)


def _encode(x, w):
    return pl.pallas_call(
        _encode_body,
        out_shape=jax.ShapeDtypeStruct((PAD_NODES, D), jnp.float32),
    )(x, w)


def _edge_agg(m, idx, zeros):
    mesh = plsc.VectorSubcoreMesh(core_axis_name="c", subcore_axis_name="s")

    @functools.partial(
        pl.kernel,
        mesh=mesh,
        out_type=jax.ShapeDtypeStruct((2, PAD_NODES, D), jnp.float32),
        scratch_types=[
            pltpu.VMEM((2, CH, EG), jnp.int32),
            pltpu.VMEM((EG, D), jnp.float32),
            pltpu.VMEM((EG, D), jnp.float32),
            pltpu.VMEM_SHARED((PAD_NODES, D), jnp.float32),
            pltpu.SemaphoreType.DMA,
            pltpu.SemaphoreType.DMA,
        ],
    )
    def body(m_hbm, idx_hbm, zero_hbm, out_hbm,
             idx_v, rows0_v, rows1_v, agg_sh, sem0, sem1):
        c = lax.axis_index("c")
        s = lax.axis_index("s")
        wid = s * 2 + c
        rows = (rows0_v, rows1_v)
        sems = (sem0, sem1)

        pltpu.sync_copy(zero_hbm.at[pl.ds(s * ROWS_PER_TILE, ROWS_PER_TILE)],
                        agg_sh.at[pl.ds(s * ROWS_PER_TILE, ROWS_PER_TILE)])
        plsc.subcore_barrier()

        def chunk_body(ch, carry):
            pltpu.sync_copy(idx_hbm.at[wid * NCH + ch], idx_v)
            pltpu.async_copy(m_hbm.at[idx_v.at[0, 0]], rows0_v, sem0)
            pltpu.async_copy(m_hbm.at[idx_v.at[0, 1]], rows1_v, sem1)

            def step(i, carry2):
                g = 2 * i
                for b in range(2):
                    pltpu.make_async_copy(
                        m_hbm.at[idx_v.at[0, g + b]], rows[b], sems[b]).wait()
                    pltpu.sync_copy(rows[b], agg_sh.at[idx_v.at[1, g + b]],
                                    add=True)

                    @pl.when(g + b + 2 < CH)
                    def _():
                        pltpu.async_copy(
                            m_hbm.at[idx_v.at[0, g + b + 2]], rows[b], sems[b])
                return carry2

            lax.fori_loop(0, CH // 2, step, 0)
            return carry

        lax.fori_loop(0, NCH, chunk_body, 0)

        plsc.subcore_barrier()
        pltpu.sync_copy(agg_sh.at[pl.ds(s * ROWS_PER_TILE, ROWS_PER_TILE)],
                        out_hbm.at[c].at[pl.ds(s * ROWS_PER_TILE, ROWS_PER_TILE)])

    return body(m, idx, zeros)


def _update_body(m_ref, a_ref, h_ref):
    m = m_ref[...]
    a = a_ref[0] + a_ref[1]
    lane = lax.broadcasted_iota(jnp.int32, m.shape, 1)
    is0 = lane == 0
    deg = jnp.sum(jnp.where(is0, a, 0.0), axis=-1, keepdims=True)
    deg = jnp.maximum(deg, 1.0)
    vs = jnp.where(is0, 0.0, m + a / deg)
    n = jnp.maximum(jnp.sqrt(jnp.sum(vs * vs, axis=-1, keepdims=True)), 1e-7)
    sinh_n = 0.5 * (jnp.exp(n) - jnp.exp(-n))
    xs = (sinh_n / n) * vs
    x0 = jnp.sqrt(1.0 + jnp.sum(xs * xs, axis=-1, keepdims=True))
    h_ref[...] = jnp.where(is0, x0, xs)


def _update(m, agg2):
    return pl.pallas_call(
        _update_body,
        out_shape=jax.ShapeDtypeStruct((PAD_NODES, D), jnp.float32),
    )(m, agg2)


def _pair_gather(h, pu, pv):
    mesh = plsc.VectorSubcoreMesh(core_axis_name="c", subcore_axis_name="s")

    @functools.partial(
        pl.kernel,
        mesh=mesh,
        out_type=jax.ShapeDtypeStruct((2, N_PAIRS, D), jnp.float32),
        scratch_types=[
            pltpu.VMEM((EG,), jnp.int32),
            pltpu.VMEM((EG, D), jnp.float32),
            pltpu.SemaphoreType.DMA,
        ],
    )
    def body(h_hbm, pu_hbm, pv_hbm, out_hbm, idx_v, rows_v, sem):
        c = lax.axis_index("c")
        s = lax.axis_index("s")
        wid = s * 2 + c
        off = wid * EG
        pltpu.sync_copy(pu_hbm.at[pl.ds(off, EG)], idx_v)
        pltpu.async_copy(h_hbm.at[idx_v], rows_v, sem).wait()
        pltpu.sync_copy(rows_v, out_hbm.at[0].at[pl.ds(off, EG)])
        pltpu.sync_copy(pv_hbm.at[pl.ds(off, EG)], idx_v)
        pltpu.async_copy(h_hbm.at[idx_v], rows_v, sem).wait()
        pltpu.sync_copy(rows_v, out_hbm.at[1].at[pl.ds(off, EG)])

    return body(h, pu, pv)


def _dist_body(g_ref, o_ref):
    eu = g_ref[0]
    ev = g_ref[1]
    p = eu * ev
    lane = lax.broadcasted_iota(jnp.int32, p.shape, 1)
    is0 = lane == 0
    p0 = jnp.sum(jnp.where(is0, p, 0.0), axis=-1, keepdims=True)
    rest = jnp.sum(jnp.where(is0, 0.0, p), axis=-1, keepdims=True)
    neg_inner = p0 - rest
    d = _acosh(jnp.maximum(neg_inner, 1.0 + 1e-5))
    o_ref[...] = jnp.broadcast_to(jnp.minimum(d * d, 15.0), p.shape)


def _dist(gath):
    return pl.pallas_call(
        _dist_body,
        out_shape=jax.ShapeDtypeStruct((N_PAIRS, D), jnp.float32),
    )(gath)


def kernel(utg, vtg, W, edge_index, pair_idx):
    x = jnp.concatenate(
        [utg, vtg, jnp.zeros((PAD_NODES - N_NODES, D), jnp.float32)], axis=0)
    m = _encode(x, W)

    n_pad_e = NE_PAD - N_EDGES
    pad = (N_NODES
           + jnp.arange(n_pad_e, dtype=jnp.int32) % (PAD_NODES - N_NODES))
    src = jnp.concatenate([edge_index[0].astype(jnp.int32), pad])
    dst = jnp.concatenate([edge_index[1].astype(jnp.int32), pad])
    idx = jnp.stack([src.reshape(NW * NCH, CH, EG),
                     dst.reshape(NW * NCH, CH, EG)], axis=1)
    zeros = jnp.zeros((PAD_NODES, D), jnp.float32)
    agg2 = _edge_agg(m, idx, zeros)

    h = _update(m, agg2)

    pu = pair_idx[:, 0].astype(jnp.int32)
    pv = pair_idx[:, 1].astype(jnp.int32)
    gath = _pair_gather(h, pu, pv)

    out = _dist(gath)
    return out[:, :1]
